# baseline (speedup 1.0000x reference)
import functools

import jax
import jax.numpy as jnp
from jax import lax
from jax.experimental import pallas as pl
from jax.experimental.pallas import tpu as pltpu
from jax.experimental.pallas import tpu_sc as plsc

N = 10000
E = 160000
D = 128
HID = 128
H = 2
B = 16384
F32 = jnp.float32

_RELS = ['binds', 'presents_to', 'contacts', 'bound_by', 'contacted_by']
_SRC = [0, 1, 0, 1, 2]
_DST = [1, 2, 2, 0, 0]


_NBLK = 1000


def _proj_body(pep, mhc, tcr, wl, bl, wr, br, xs, xd):
    embs = (pep[...], mhc[...], tcr[...])
    for r in range(5):
        for h in range(2):
            xs[r, h] = (jnp.dot(embs[_SRC[r]], wl[r, h],
                                preferred_element_type=F32)
                        + bl[2 * r + h][None, :])
            xd[r, h] = (jnp.dot(embs[_DST[r]], wr[r, h],
                                preferred_element_type=F32)
                        + br[2 * r + h][None, :])


def _proj_call(pep, mhc, tcr, wl, bl, wr, br):
    grid = (N // _NBLK,)
    node_spec = pl.BlockSpec((_NBLK, D), lambda i: (i, 0))
    full4 = pl.BlockSpec((5, 2, D, D), lambda i: (0, 0, 0, 0))
    full2 = pl.BlockSpec((10, D), lambda i: (0, 0))
    out_spec = pl.BlockSpec((5, 2, _NBLK, D), lambda i: (0, 0, i, 0))
    return pl.pallas_call(
        _proj_body,
        grid=grid,
        in_specs=[node_spec, node_spec, node_spec, full4, full2, full4, full2],
        out_specs=[out_spec, out_spec],
        out_shape=[jax.ShapeDtypeStruct((5, 2, N, D), F32)] * 2,
    )(pep, mhc, tcr, wl, bl, wr, br)


_EC = 40
_NCHUNK = 252
_EPAD = 16 * _NCHUNK * _EC - E
_ROWS_T = N // 16
_DROWS = 25
_W = 144
_NSP = N + 16
_PB = 48


def _edge_body(xs_flat, xd_flat, edges7, att2, out_flat,
               num_sp, ib0, ib1, ib2, ib3, rs_a, rs_b, rd_a, rd_b,
               ob_a, ob_b, attv, nbuf, pbuf, exbuf,
               sem_a, sem_b, ss_a, ss_b):
    c = lax.axis_index("c")
    t = lax.axis_index("s")
    z16 = jnp.zeros((16,), F32)
    mask0 = lax.iota(jnp.int32, 16) == 0
    iota16 = lax.iota(jnp.int32, 16)
    ibs = [ib0, ib1, ib2, ib3]
    rss = [rs_a, rs_b]
    rds = [rd_a, rd_b]
    obs = [ob_a, ob_b]
    sems = [sem_a, sem_b]
    sss = [ss_a, ss_b]

    for k in range(_EC, _PB):
        pbuf[k] = z16

    def relation(r, carry0):
        def zrow(i, carry):
            for j in range(_W // 16):
                ob_a[i, 16 * j:16 * (j + 1)] = z16
            return carry

        lax.fori_loop(0, _DROWS, zrow, 0)

        def zcopy(k, carry):
            pltpu.sync_copy(ob_a.at[pl.ds(0, _DROWS)],
                            num_sp.at[pl.ds(t * _ROWS_T + k * _DROWS,
                                            _DROWS)])
            return carry

        lax.fori_loop(0, _ROWS_T // _DROWS, zcopy, 0)
        plsc.subcore_barrier()
        pltpu.sync_copy(att2.at[2 * r + c], attv)
        atts = [attv[16 * j:16 * (j + 1)] for j in range(8)]

        def issue(k, ib, rs, rd, sem):
            pltpu.sync_copy(edges7.at[r, c, t * _NCHUNK + k], ib)
            pltpu.async_copy(xs_flat.at[ib.at[0]], rs, sem)
            pltpu.async_copy(xd_flat.at[ib.at[1]], rd, sem)

        def wait(ib, rs, rd, sem):
            pltpu.make_async_copy(xs_flat.at[ib.at[0]], rs, sem).wait()
            pltpu.make_async_copy(xd_flat.at[ib.at[1]], rd, sem).wait()

        def compute(rs, rd, ob):
            def edge1(e, ecarry):
                acc = z16
                for j in range(8):
                    sl = slice(16 * j, 16 * (j + 1))
                    x = rs[e, sl] + rd[e, sl]
                    lr = jnp.where(x >= 0.0, x, 0.2 * x)
                    acc = acc + lr * atts[j]
                pbuf[e] = acc
                return ecarry

            lax.fori_loop(0, _EC, edge1, 0)

            for g in range(_PB // 16):
                rowi = g * 16 + iota16
                cols = [plsc.load_gather(
                    pbuf, [rowi, jnp.full((16,), j, jnp.int32)])
                    for j in range(16)]
                while len(cols) > 1:
                    cols = [a + b for a, b in zip(cols[::2], cols[1::2])]
                exbuf[16 * g:16 * (g + 1)] = jnp.exp(cols[0])

            def edge3(e, ecarry):
                exr = plsc.load_gather(exbuf,
                                       [jnp.full((16,), e, jnp.int32)])
                for j in range(8):
                    sl = slice(16 * j, 16 * (j + 1))
                    ob[e, sl] = exr * rs[e, sl]
                ob[e, 128:144] = jnp.where(mask0, exr, z16)
                return ecarry

            lax.fori_loop(0, _EC, edge3, 0)

        issue(0, ib0, rs_a, rd_a, sem_a)

        def quad(i, carry):
            for j in range(4):
                k = 4 * i + j
                nb = (j + 1) % 4
                issue(jnp.minimum(k + 1, _NCHUNK - 1),
                      ibs[nb], rss[nb % 2], rds[nb % 2], sems[nb % 2])
                wait(ibs[j], rss[j % 2], rds[j % 2], sems[j % 2])
                sdesc = pltpu.make_async_copy(obs[j % 2],
                                              num_sp.at[ibs[j].at[2]],
                                              sss[j % 2])
                if j >= 2:
                    sdesc.wait()
                else:
                    @pl.when(i > 0)
                    def _():
                        sdesc.wait()

                compute(rss[j % 2], rds[j % 2], obs[j % 2])
                pltpu.async_copy(obs[j % 2], num_sp.at[ibs[j].at[2]],
                                 sss[j % 2], add=True)
            return carry

        lax.fori_loop(0, _NCHUNK // 4, quad, 0)
        wait(ib0, rs_a, rd_a, sem_a)
        pltpu.make_async_copy(ob_a, num_sp.at[ib2.at[2]], ss_a).wait()
        pltpu.make_async_copy(ob_b, num_sp.at[ib3.at[2]], ss_b).wait()
        plsc.subcore_barrier()

        def dump(k, carry):
            rb = t * _ROWS_T + k * _DROWS
            pltpu.sync_copy(num_sp.at[pl.ds(rb, _DROWS)],
                            ob_a.at[pl.ds(0, _DROWS)])

            def nrow(i, ncarry):
                den = ob_a[i, 128:144][0]
                inv = 1.0 / (jnp.full((16,), den) + 1e-16)
                for j in range(8):
                    nbuf[i, 16 * j:16 * (j + 1)] = \
                        ob_a[i, 16 * j:16 * (j + 1)] * inv
                return ncarry

            lax.fori_loop(0, _DROWS, nrow, 0)
            pltpu.sync_copy(nbuf,
                            out_flat.at[pl.ds((2 * r + c) * N + rb, _DROWS)])
            return carry

        lax.fori_loop(0, _ROWS_T // _DROWS, dump, 0)
        plsc.subcore_barrier()
        return carry0

    lax.fori_loop(0, 5, relation, 0)


def _edge_call(xs_flat, xd_flat, edges7, att2):
    mesh = plsc.VectorSubcoreMesh(core_axis_name="c", subcore_axis_name="s")
    f = pl.kernel(
        _edge_body,
        out_type=jax.ShapeDtypeStruct((10 * N, D), F32),
        mesh=mesh,
        compiler_params=pltpu.CompilerParams(use_tc_tiling_on_sc=False, needs_layout_passes=False),
        scratch_types=[
            pltpu.VMEM_SHARED((_NSP, _W), F32),
            pltpu.VMEM((3, _EC), jnp.int32),
            pltpu.VMEM((3, _EC), jnp.int32),
            pltpu.VMEM((3, _EC), jnp.int32),
            pltpu.VMEM((3, _EC), jnp.int32),
            pltpu.VMEM((_EC, D), F32),
            pltpu.VMEM((_EC, D), F32),
            pltpu.VMEM((_EC, D), F32),
            pltpu.VMEM((_EC, D), F32),
            pltpu.VMEM((_EC, _W), F32),
            pltpu.VMEM((_EC, _W), F32),
            pltpu.VMEM((D,), F32),
            pltpu.VMEM((_DROWS, D), F32),
            pltpu.VMEM((_PB, 16), F32),
            pltpu.VMEM((_PB,), F32),
            pltpu.SemaphoreType.DMA,
            pltpu.SemaphoreType.DMA,
            pltpu.SemaphoreType.DMA,
            pltpu.SemaphoreType.DMA,
        ],
    )
    return f(xs_flat, xd_flat, edges7, att2)


def _combine_body(num, pep, mhc, tcr, rb, wp, bp, h_all):
    def rel(r):
        return 0.5 * (num[r, 0] + num[r, 1]) + rb[r][None, :]

    def elu(x):
        return jnp.where(x > 0.0, x, jnp.exp(jnp.minimum(x, 0.0)) - 1.0)

    out_mhc = rel(0)
    out_tcr = 0.5 * (rel(1) + rel(2))
    out_pep = 0.5 * (rel(3) + rel(4))
    h_pep = elu(out_pep + pep[...])
    h_all[0] = jnp.dot(h_pep, wp[...], preferred_element_type=F32) \
        + bp[0][None, :]
    h_all[1] = elu(out_mhc + mhc[...])
    h_all[2] = elu(out_tcr + tcr[...])


def _combine_call(num4, pep, mhc, tcr, rel_bias, wp, bp):
    grid = (N // _NBLK,)
    node_spec = pl.BlockSpec((_NBLK, D), lambda i: (i, 0))
    return pl.pallas_call(
        _combine_body,
        grid=grid,
        in_specs=[
            pl.BlockSpec((5, 2, _NBLK, D), lambda i: (0, 0, i, 0)),
            node_spec, node_spec, node_spec,
            pl.BlockSpec((5, D), lambda i: (0, 0)),
            pl.BlockSpec((D, D), lambda i: (0, 0)),
            pl.BlockSpec((1, D), lambda i: (0, 0)),
        ],
        out_specs=pl.BlockSpec((3, _NBLK, D), lambda i: (0, i, 0)),
        out_shape=jax.ShapeDtypeStruct((3, N, D), F32),
    )(num4, pep, mhc, tcr, rel_bias, wp, bp)


_GC = 128
_GPW = 3 * B // 32 // _GC


def _tgather_body(table, tidx, out, ibuf, rbuf):
    c = lax.axis_index("c")
    t = lax.axis_index("s")
    wid = t * 2 + c

    def chunk(k, carry):
        base = wid * (_GPW * _GC) + k * _GC
        pltpu.sync_copy(tidx.at[pl.ds(base, _GC)], ibuf)
        pltpu.sync_copy(table.at[ibuf], rbuf)
        pltpu.sync_copy(rbuf, out.at[pl.ds(base, _GC)])
        return carry

    lax.fori_loop(0, _GPW, chunk, 0)


def _tgather_call(table_flat, tidx_flat):
    mesh = plsc.VectorSubcoreMesh(core_axis_name="c", subcore_axis_name="s")
    f = pl.kernel(
        _tgather_body,
        out_type=jax.ShapeDtypeStruct((3 * B, D), F32),
        mesh=mesh,
        compiler_params=pltpu.CompilerParams(use_tc_tiling_on_sc=False, needs_layout_passes=False),
        scratch_types=[
            pltpu.VMEM((_GC,), jnp.int32),
            pltpu.VMEM((_GC, D), F32),
        ],
    )
    return f(table_flat, tidx_flat)


_BBLK = 1024


def _mlp_body(hb, w1pm, b1pm, w2pm, b2pm, wpm, w1mt, b1mt, w2mt, b2mt,
              w1df, b1df, wdf2, scb, out):
    hpb, hmb, htb = hb[0], hb[1], hb[2]

    def mm(x, w):
        return jnp.dot(x, w, preferred_element_type=F32)

    v = jnp.maximum(mm(hpb, w1pm[:D]) + mm(hmb, w1pm[D:]) + b1pm[0][None, :],
                    0.0)
    v_pm = mm(v, w2pm[...]) + b2pm[0][None, :]
    logit_pm = jnp.sum(v_pm * wpm[0][None, :], axis=1) + scb[0, 0]
    u = jnp.maximum(mm(hmb, w1mt[:D]) + mm(htb, w1mt[D:]) + b1mt[0][None, :],
                    0.0)
    v_mt = mm(u, w2mt[...]) + b2mt[0][None, :]
    z = v_pm * v_mt
    z1 = jnp.maximum(mm(z, w1df[...]) + b1df[0][None, :], 0.0)
    logit_pmt = jnp.sum(z1 * wdf2[0][None, :], axis=1) + scb[1, 0]
    out[0] = logit_pm
    out[1] = logit_pmt


def _mlp_call(hb3, p):
    grid = (B // _BBLK,)

    def full(shape):
        nd = len(shape)
        return pl.BlockSpec(shape, lambda i, _n=nd: (0,) * _n)

    w1pm = p['f_pm']['l1']['W'].T
    b1pm = p['f_pm']['l1']['b'][None, :]
    w2pm = p['f_pm']['l2']['W'].T
    b2pm = p['f_pm']['l2']['b'][None, :]
    wpm = p['w_pm']['W']
    w1mt = p['f_mt']['l1']['W'].T
    b1mt = p['f_mt']['l1']['b'][None, :]
    w2mt = p['f_mt']['l2']['W'].T
    b2mt = p['f_mt']['l2']['b'][None, :]
    w1df = p['f_dmf']['l1']['W'].T
    b1df = p['f_dmf']['l1']['b'][None, :]
    wdf2 = p['f_dmf']['l2']['W']
    scb = jnp.stack([
        jnp.pad(p['w_pm']['b'], (0, D - 1)),
        jnp.pad(p['f_dmf']['l2']['b'], (0, D - 1)),
    ])
    return pl.pallas_call(
        _mlp_body,
        grid=grid,
        in_specs=[
            pl.BlockSpec((3, _BBLK, D), lambda i: (0, i, 0)),
            full((2 * D, D)), full((1, D)), full((D, D)), full((1, D)),
            full((1, D)),
            full((2 * D, D)), full((1, D)), full((D, D)), full((1, D)),
            full((D, D)), full((1, D)), full((1, D)), full((2, D)),
        ],
        out_specs=pl.BlockSpec((2, _BBLK), lambda i: (0, i)),
        out_shape=jax.ShapeDtypeStruct((2, B), F32),
    )(hb3, w1pm, b1pm, w2pm, b2pm, wpm, w1mt, b1mt, w2mt, b2mt,
      w1df, b1df, wdf2, scb)


def kernel(params, edge_binds, edge_presents_to, edge_contacts,
           edge_bound_by, edge_contacted_by, triplet_idx):
    p = params
    rels = p['rels']

    def heads_t(w):
        return w.reshape(H, HID, D).transpose(0, 2, 1)

    wl = jnp.stack([heads_t(rels[r]['lin_l']['W']) for r in _RELS])
    wr = jnp.stack([heads_t(rels[r]['lin_r']['W']) for r in _RELS])
    bl = jnp.stack([rels[r]['lin_l']['b'] for r in _RELS]).reshape(10, D)
    br = jnp.stack([rels[r]['lin_r']['b'] for r in _RELS]).reshape(10, D)
    att2 = jnp.stack([rels[r]['att'] for r in _RELS]).reshape(10, D)
    rel_bias = jnp.stack([rels[r]['bias'] for r in _RELS])

    xs4, xd4 = _proj_call(p['emb_pep'], p['emb_mhc'], p['emb_tcr'],
                          wl, bl, wr, br)
    xs_flat = xs4.reshape(10 * N, D)
    xd_flat = xd4.reshape(10 * N, D)

    edges = [edge_binds, edge_presents_to, edge_contacts, edge_bound_by,
             edge_contacted_by]
    e_raw = jnp.stack(edges)
    offs = (jnp.arange(5, dtype=jnp.int32) * 2)[:, None, None]
    head = jnp.arange(2, dtype=jnp.int32)[None, :, None]
    src_flat = (offs + head) * N + e_raw[:, None, 0, :]
    dst_flat = (offs + head) * N + e_raw[:, None, 1, :]
    dst_loc = jnp.broadcast_to(e_raw[:, None, 1, :], (5, 2, E))
    base = ((offs + head) * N).astype(jnp.int32)
    park = jnp.arange(_EPAD, dtype=jnp.int32)[None, None, :]
    gpad = jnp.broadcast_to(base + park % 128, (5, 2, _EPAD))
    spad = jnp.broadcast_to(N + park % 16, (5, 2, _EPAD))
    src_flat = jnp.concatenate([src_flat, gpad], axis=-1)
    dst_flat = jnp.concatenate([dst_flat, gpad], axis=-1)
    dst_loc = jnp.concatenate([dst_loc, spad], axis=-1)
    edges7 = jnp.stack([x.reshape(5, 2, 16 * _NCHUNK, _EC)
                        for x in (src_flat, dst_flat, dst_loc)], axis=3)

    num_flat = _edge_call(xs_flat, xd_flat, edges7, att2)
    num4 = num_flat.reshape(5, 2, N, D)

    h_all = _combine_call(num4, p['emb_pep'], p['emb_mhc'], p['emb_tcr'],
                          rel_bias, p['proj_pep']['W'].T,
                          p['proj_pep']['b'][None, :])
    table_flat = h_all.reshape(3 * N, D)

    tidx_flat = (triplet_idx
                 + (jnp.arange(3, dtype=jnp.int32) * N)[:, None]).reshape(-1)
    hb_flat = _tgather_call(table_flat, tidx_flat)
    hb3 = hb_flat.reshape(3, B, D)

    return _mlp_call(hb3, p)


# speedup vs baseline: 1.3362x; 1.3362x over previous
import functools

import jax
import jax.numpy as jnp
from jax import lax
from jax.experimental import pallas as pl
from jax.experimental.pallas import tpu as pltpu
from jax.experimental.pallas import tpu_sc as plsc

N = 10000
E = 160000
D = 128
HID = 128
H = 2
B = 16384
F32 = jnp.float32

_RELS = ['binds', 'presents_to', 'contacts', 'bound_by', 'contacted_by']
_SRC = [0, 1, 0, 1, 2]
_DST = [1, 2, 2, 0, 0]


_NBLK = 1000


def _proj_body(pep, mhc, tcr, wl, bl, wr, br, xs, xd):
    embs = (pep[...], mhc[...], tcr[...])
    for r in range(5):
        for h in range(2):
            xs[r, h] = (jnp.dot(embs[_SRC[r]], wl[r, h],
                                preferred_element_type=F32)
                        + bl[2 * r + h][None, :])
            xd[r, h] = (jnp.dot(embs[_DST[r]], wr[r, h],
                                preferred_element_type=F32)
                        + br[2 * r + h][None, :])


def _proj_call(pep, mhc, tcr, wl, bl, wr, br):
    grid = (N // _NBLK,)
    node_spec = pl.BlockSpec((_NBLK, D), lambda i: (i, 0))
    full4 = pl.BlockSpec((5, 2, D, D), lambda i: (0, 0, 0, 0))
    full2 = pl.BlockSpec((10, D), lambda i: (0, 0))
    out_spec = pl.BlockSpec((5, 2, _NBLK, D), lambda i: (0, 0, i, 0))
    return pl.pallas_call(
        _proj_body,
        grid=grid,
        in_specs=[node_spec, node_spec, node_spec, full4, full2, full4, full2],
        out_specs=[out_spec, out_spec],
        out_shape=[jax.ShapeDtypeStruct((5, 2, N, D), F32)] * 2,
    )(pep, mhc, tcr, wl, bl, wr, br)


_EC = 40
_NCHUNK = 252
_EPAD = 16 * _NCHUNK * _EC - E
_ROWS_T = N // 16
_DROWS = 25
_W = 144
_NSP = N + 16
_PB = 48


def _edge_body(xs_flat, xd_flat, edges7, att2, out_flat,
               num_sp, ib0, ib1, ib2, ib3, rs_a, rs_b, rd_a, rd_b,
               ob_a, ob_b, attv, nbuf,
               sem_a, sem_b, ss_a, ss_b):
    c = lax.axis_index("c")
    t = lax.axis_index("s")
    z16 = jnp.zeros((16,), F32)
    mask0 = lax.iota(jnp.int32, 16) == 0
    iota16 = lax.iota(jnp.int32, 16)
    ibs = [ib0, ib1, ib2, ib3]
    rss = [rs_a, rs_b]
    rds = [rd_a, rd_b]
    obs = [ob_a, ob_b]
    sems = [sem_a, sem_b]
    sss = [ss_a, ss_b]

    def relation(r, carry0):
        def zrow(i, carry):
            for j in range(_W // 16):
                ob_a[i, 16 * j:16 * (j + 1)] = z16
            return carry

        lax.fori_loop(0, _DROWS, zrow, 0)

        def zcopy(k, carry):
            pltpu.sync_copy(ob_a.at[pl.ds(0, _DROWS)],
                            num_sp.at[pl.ds(t * _ROWS_T + k * _DROWS,
                                            _DROWS)])
            return carry

        lax.fori_loop(0, _ROWS_T // _DROWS, zcopy, 0)
        plsc.subcore_barrier()
        pltpu.sync_copy(att2.at[2 * r + c], attv)
        atts = [attv[16 * j:16 * (j + 1)] for j in range(8)]

        def issue(k, ib, rs, rd, sem):
            pltpu.sync_copy(edges7.at[r, c, t * _NCHUNK + k], ib)
            pltpu.async_copy(xs_flat.at[ib.at[0]], rs, sem)
            pltpu.async_copy(xd_flat.at[ib.at[1]], rd, sem)

        def wait(ib, rs, rd, sem):
            pltpu.make_async_copy(xs_flat.at[ib.at[0]], rs, sem).wait()
            pltpu.make_async_copy(xd_flat.at[ib.at[1]], rd, sem).wait()

        def compute(rs, rd, ob):
            def edge2(i, ecarry):
                for e in (2 * i, 2 * i + 1):
                    svals = [rs[e, 16 * j:16 * (j + 1)] for j in range(8)]
                    x0 = svals[0] + rd[e, 0:16]
                    acc = jnp.where(x0 >= 0.0, x0, 0.2 * x0) * atts[0]
                    for j in range(1, 8):
                        x = svals[j] + rd[e, 16 * j:16 * (j + 1)]
                        lr = jnp.where(x >= 0.0, x, 0.2 * x)
                        acc = acc + lr * atts[j]
                    exv = jnp.exp(jnp.full((16,), jnp.sum(acc)))
                    for j in range(8):
                        ob[e, 16 * j:16 * (j + 1)] = exv * svals[j]
                    ob[e, 128:144] = jnp.where(mask0, exv, z16)
                return ecarry

            lax.fori_loop(0, _EC // 2, edge2, 0)

        issue(0, ib0, rs_a, rd_a, sem_a)

        def quad(i, carry):
            for j in range(4):
                k = 4 * i + j
                nb = (j + 1) % 4
                issue(jnp.minimum(k + 1, _NCHUNK - 1),
                      ibs[nb], rss[nb % 2], rds[nb % 2], sems[nb % 2])
                wait(ibs[j], rss[j % 2], rds[j % 2], sems[j % 2])
                sdesc = pltpu.make_async_copy(obs[j % 2],
                                              num_sp.at[ibs[j].at[2]],
                                              sss[j % 2])
                if j >= 2:
                    sdesc.wait()
                else:
                    @pl.when(i > 0)
                    def _():
                        sdesc.wait()

                compute(rss[j % 2], rds[j % 2], obs[j % 2])
                pltpu.async_copy(obs[j % 2], num_sp.at[ibs[j].at[2]],
                                 sss[j % 2], add=True)
            return carry

        lax.fori_loop(0, _NCHUNK // 4, quad, 0)
        wait(ib0, rs_a, rd_a, sem_a)
        pltpu.make_async_copy(ob_a, num_sp.at[ib2.at[2]], ss_a).wait()
        pltpu.make_async_copy(ob_b, num_sp.at[ib3.at[2]], ss_b).wait()
        plsc.subcore_barrier()

        def dump(k, carry):
            rb = t * _ROWS_T + k * _DROWS
            pltpu.sync_copy(num_sp.at[pl.ds(rb, _DROWS)],
                            ob_a.at[pl.ds(0, _DROWS)])

            def nrow(i, ncarry):
                den = ob_a[i, 128:144][0]
                inv = 1.0 / (jnp.full((16,), den) + 1e-16)
                for j in range(8):
                    nbuf[i, 16 * j:16 * (j + 1)] = \
                        ob_a[i, 16 * j:16 * (j + 1)] * inv
                return ncarry

            lax.fori_loop(0, _DROWS, nrow, 0)
            pltpu.sync_copy(nbuf,
                            out_flat.at[pl.ds((2 * r + c) * N + rb, _DROWS)])
            return carry

        lax.fori_loop(0, _ROWS_T // _DROWS, dump, 0)
        plsc.subcore_barrier()
        return carry0

    lax.fori_loop(0, 5, relation, 0)


def _edge_call(xs_flat, xd_flat, edges7, att2):
    mesh = plsc.VectorSubcoreMesh(core_axis_name="c", subcore_axis_name="s")
    f = pl.kernel(
        _edge_body,
        out_type=jax.ShapeDtypeStruct((10 * N, D), F32),
        mesh=mesh,
        compiler_params=pltpu.CompilerParams(use_tc_tiling_on_sc=False, needs_layout_passes=False),
        scratch_types=[
            pltpu.VMEM_SHARED((_NSP, _W), F32),
            pltpu.VMEM((3, _EC), jnp.int32),
            pltpu.VMEM((3, _EC), jnp.int32),
            pltpu.VMEM((3, _EC), jnp.int32),
            pltpu.VMEM((3, _EC), jnp.int32),
            pltpu.VMEM((_EC, D), F32),
            pltpu.VMEM((_EC, D), F32),
            pltpu.VMEM((_EC, D), F32),
            pltpu.VMEM((_EC, D), F32),
            pltpu.VMEM((_EC, _W), F32),
            pltpu.VMEM((_EC, _W), F32),
            pltpu.VMEM((D,), F32),
            pltpu.VMEM((_DROWS, D), F32),
            pltpu.SemaphoreType.DMA,
            pltpu.SemaphoreType.DMA,
            pltpu.SemaphoreType.DMA,
            pltpu.SemaphoreType.DMA,
        ],
    )
    return f(xs_flat, xd_flat, edges7, att2)


def _combine_body(num, pep, mhc, tcr, rb, wp, bp, h_all):
    def rel(r):
        return 0.5 * (num[r, 0] + num[r, 1]) + rb[r][None, :]

    def elu(x):
        return jnp.where(x > 0.0, x, jnp.exp(jnp.minimum(x, 0.0)) - 1.0)

    out_mhc = rel(0)
    out_tcr = 0.5 * (rel(1) + rel(2))
    out_pep = 0.5 * (rel(3) + rel(4))
    h_pep = elu(out_pep + pep[...])
    h_all[0] = jnp.dot(h_pep, wp[...], preferred_element_type=F32) \
        + bp[0][None, :]
    h_all[1] = elu(out_mhc + mhc[...])
    h_all[2] = elu(out_tcr + tcr[...])


def _combine_call(num4, pep, mhc, tcr, rel_bias, wp, bp):
    grid = (N // _NBLK,)
    node_spec = pl.BlockSpec((_NBLK, D), lambda i: (i, 0))
    return pl.pallas_call(
        _combine_body,
        grid=grid,
        in_specs=[
            pl.BlockSpec((5, 2, _NBLK, D), lambda i: (0, 0, i, 0)),
            node_spec, node_spec, node_spec,
            pl.BlockSpec((5, D), lambda i: (0, 0)),
            pl.BlockSpec((D, D), lambda i: (0, 0)),
            pl.BlockSpec((1, D), lambda i: (0, 0)),
        ],
        out_specs=pl.BlockSpec((3, _NBLK, D), lambda i: (0, i, 0)),
        out_shape=jax.ShapeDtypeStruct((3, N, D), F32),
    )(num4, pep, mhc, tcr, rel_bias, wp, bp)


_GC = 128
_GPW = 3 * B // 32 // _GC


def _tgather_body(table, tidx, out, ibuf, rbuf):
    c = lax.axis_index("c")
    t = lax.axis_index("s")
    wid = t * 2 + c

    def chunk(k, carry):
        base = wid * (_GPW * _GC) + k * _GC
        pltpu.sync_copy(tidx.at[pl.ds(base, _GC)], ibuf)
        pltpu.sync_copy(table.at[ibuf], rbuf)
        pltpu.sync_copy(rbuf, out.at[pl.ds(base, _GC)])
        return carry

    lax.fori_loop(0, _GPW, chunk, 0)


def _tgather_call(table_flat, tidx_flat):
    mesh = plsc.VectorSubcoreMesh(core_axis_name="c", subcore_axis_name="s")
    f = pl.kernel(
        _tgather_body,
        out_type=jax.ShapeDtypeStruct((3 * B, D), F32),
        mesh=mesh,
        compiler_params=pltpu.CompilerParams(use_tc_tiling_on_sc=False, needs_layout_passes=False),
        scratch_types=[
            pltpu.VMEM((_GC,), jnp.int32),
            pltpu.VMEM((_GC, D), F32),
        ],
    )
    return f(table_flat, tidx_flat)


_BBLK = 1024


def _mlp_body(hb, w1pm, b1pm, w2pm, b2pm, wpm, w1mt, b1mt, w2mt, b2mt,
              w1df, b1df, wdf2, scb, out):
    hpb, hmb, htb = hb[0], hb[1], hb[2]

    def mm(x, w):
        return jnp.dot(x, w, preferred_element_type=F32)

    v = jnp.maximum(mm(hpb, w1pm[:D]) + mm(hmb, w1pm[D:]) + b1pm[0][None, :],
                    0.0)
    v_pm = mm(v, w2pm[...]) + b2pm[0][None, :]
    logit_pm = jnp.sum(v_pm * wpm[0][None, :], axis=1) + scb[0, 0]
    u = jnp.maximum(mm(hmb, w1mt[:D]) + mm(htb, w1mt[D:]) + b1mt[0][None, :],
                    0.0)
    v_mt = mm(u, w2mt[...]) + b2mt[0][None, :]
    z = v_pm * v_mt
    z1 = jnp.maximum(mm(z, w1df[...]) + b1df[0][None, :], 0.0)
    logit_pmt = jnp.sum(z1 * wdf2[0][None, :], axis=1) + scb[1, 0]
    out[0] = logit_pm
    out[1] = logit_pmt


def _mlp_call(hb3, p):
    grid = (B // _BBLK,)

    def full(shape):
        nd = len(shape)
        return pl.BlockSpec(shape, lambda i, _n=nd: (0,) * _n)

    w1pm = p['f_pm']['l1']['W'].T
    b1pm = p['f_pm']['l1']['b'][None, :]
    w2pm = p['f_pm']['l2']['W'].T
    b2pm = p['f_pm']['l2']['b'][None, :]
    wpm = p['w_pm']['W']
    w1mt = p['f_mt']['l1']['W'].T
    b1mt = p['f_mt']['l1']['b'][None, :]
    w2mt = p['f_mt']['l2']['W'].T
    b2mt = p['f_mt']['l2']['b'][None, :]
    w1df = p['f_dmf']['l1']['W'].T
    b1df = p['f_dmf']['l1']['b'][None, :]
    wdf2 = p['f_dmf']['l2']['W']
    scb = jnp.stack([
        jnp.pad(p['w_pm']['b'], (0, D - 1)),
        jnp.pad(p['f_dmf']['l2']['b'], (0, D - 1)),
    ])
    return pl.pallas_call(
        _mlp_body,
        grid=grid,
        in_specs=[
            pl.BlockSpec((3, _BBLK, D), lambda i: (0, i, 0)),
            full((2 * D, D)), full((1, D)), full((D, D)), full((1, D)),
            full((1, D)),
            full((2 * D, D)), full((1, D)), full((D, D)), full((1, D)),
            full((D, D)), full((1, D)), full((1, D)), full((2, D)),
        ],
        out_specs=pl.BlockSpec((2, _BBLK), lambda i: (0, i)),
        out_shape=jax.ShapeDtypeStruct((2, B), F32),
    )(hb3, w1pm, b1pm, w2pm, b2pm, wpm, w1mt, b1mt, w2mt, b2mt,
      w1df, b1df, wdf2, scb)


def kernel(params, edge_binds, edge_presents_to, edge_contacts,
           edge_bound_by, edge_contacted_by, triplet_idx):
    p = params
    rels = p['rels']

    def heads_t(w):
        return w.reshape(H, HID, D).transpose(0, 2, 1)

    wl = jnp.stack([heads_t(rels[r]['lin_l']['W']) for r in _RELS])
    wr = jnp.stack([heads_t(rels[r]['lin_r']['W']) for r in _RELS])
    bl = jnp.stack([rels[r]['lin_l']['b'] for r in _RELS]).reshape(10, D)
    br = jnp.stack([rels[r]['lin_r']['b'] for r in _RELS]).reshape(10, D)
    att2 = jnp.stack([rels[r]['att'] for r in _RELS]).reshape(10, D)
    rel_bias = jnp.stack([rels[r]['bias'] for r in _RELS])

    xs4, xd4 = _proj_call(p['emb_pep'], p['emb_mhc'], p['emb_tcr'],
                          wl, bl, wr, br)
    xs_flat = xs4.reshape(10 * N, D)
    xd_flat = xd4.reshape(10 * N, D)

    edges = [edge_binds, edge_presents_to, edge_contacts, edge_bound_by,
             edge_contacted_by]
    e_raw = jnp.stack(edges)
    offs = (jnp.arange(5, dtype=jnp.int32) * 2)[:, None, None]
    head = jnp.arange(2, dtype=jnp.int32)[None, :, None]
    src_flat = (offs + head) * N + e_raw[:, None, 0, :]
    dst_flat = (offs + head) * N + e_raw[:, None, 1, :]
    dst_loc = jnp.broadcast_to(e_raw[:, None, 1, :], (5, 2, E))
    base = ((offs + head) * N).astype(jnp.int32)
    park = jnp.arange(_EPAD, dtype=jnp.int32)[None, None, :]
    gpad = jnp.broadcast_to(base + park % 128, (5, 2, _EPAD))
    spad = jnp.broadcast_to(N + park % 16, (5, 2, _EPAD))
    src_flat = jnp.concatenate([src_flat, gpad], axis=-1)
    dst_flat = jnp.concatenate([dst_flat, gpad], axis=-1)
    dst_loc = jnp.concatenate([dst_loc, spad], axis=-1)
    edges7 = jnp.stack([x.reshape(5, 2, 16 * _NCHUNK, _EC)
                        for x in (src_flat, dst_flat, dst_loc)], axis=3)

    num_flat = _edge_call(xs_flat, xd_flat, edges7, att2)
    num4 = num_flat.reshape(5, 2, N, D)

    h_all = _combine_call(num4, p['emb_pep'], p['emb_mhc'], p['emb_tcr'],
                          rel_bias, p['proj_pep']['W'].T,
                          p['proj_pep']['b'][None, :])
    table_flat = h_all.reshape(3 * N, D)

    tidx_flat = (triplet_idx
                 + (jnp.arange(3, dtype=jnp.int32) * N)[:, None]).reshape(-1)
    hb_flat = _tgather_call(table_flat, tidx_flat)
    hb3 = hb_flat.reshape(3, B, D)

    return _mlp_call(hb3, p)


# speedup vs baseline: 1.7904x; 1.3400x over previous
import functools

import jax
import jax.numpy as jnp
from jax import lax
from jax.experimental import pallas as pl
from jax.experimental.pallas import tpu as pltpu
from jax.experimental.pallas import tpu_sc as plsc

N = 10000
E = 160000
D = 128
HID = 128
H = 2
B = 16384
F32 = jnp.float32

_RELS = ['binds', 'presents_to', 'contacts', 'bound_by', 'contacted_by']
_SRC = [0, 1, 0, 1, 2]
_DST = [1, 2, 2, 0, 0]


_NBLK = 1000


def _proj_body(pep, mhc, tcr, wl, bl, wr, br, xs, xd):
    embs = (pep[...], mhc[...], tcr[...])
    for r in range(5):
        for h in range(2):
            xs[r, h] = (jnp.dot(embs[_SRC[r]], wl[r, h],
                                preferred_element_type=F32)
                        + bl[2 * r + h][None, :])
            xd[r, h] = (jnp.dot(embs[_DST[r]], wr[r, h],
                                preferred_element_type=F32)
                        + br[2 * r + h][None, :])


def _proj_call(pep, mhc, tcr, wl, bl, wr, br):
    grid = (N // _NBLK,)
    node_spec = pl.BlockSpec((_NBLK, D), lambda i: (i, 0))
    full4 = pl.BlockSpec((5, 2, D, D), lambda i: (0, 0, 0, 0))
    full2 = pl.BlockSpec((10, D), lambda i: (0, 0))
    out_spec = pl.BlockSpec((5, 2, _NBLK, D), lambda i: (0, 0, i, 0))
    return pl.pallas_call(
        _proj_body,
        grid=grid,
        in_specs=[node_spec, node_spec, node_spec, full4, full2, full4, full2],
        out_specs=[out_spec, out_spec],
        out_shape=[jax.ShapeDtypeStruct((5, 2, N, D), F32)] * 2,
    )(pep, mhc, tcr, wl, bl, wr, br)


_EC = 40
_NCHUNK = 252
_EPAD = 16 * _NCHUNK * _EC - E
_ROWS_T = N // 16
_DROWS = 25
_W = 144
_NSP = N + 16
_PB = 48


def _edge_body(xs_flat, xd_flat, edges7, att2, out_flat,
               num_sp, ib0, ib1, ib2, ib3, rs_a, rs_b, rd_a, rd_b,
               ob_a, ob_b, attv, nbuf,
               sem_a, sem_b, ss_a, ss_b):
    c = lax.axis_index("c")
    t = lax.axis_index("s")
    z16 = jnp.zeros((16,), F32)
    mask0 = lax.iota(jnp.int32, 16) == 0
    iota16 = lax.iota(jnp.int32, 16)
    ibs = [ib0, ib1, ib2, ib3]
    rss = [rs_a, rs_b]
    rds = [rd_a, rd_b]
    obs = [ob_a, ob_b]
    sems = [sem_a, sem_b]
    sss = [ss_a, ss_b]

    def relation(r, carry0):
        def zrow(i, carry):
            for j in range(_W // 16):
                ob_a[i, 16 * j:16 * (j + 1)] = z16
            return carry

        lax.fori_loop(0, _DROWS, zrow, 0)

        def zcopy(k, carry):
            pltpu.sync_copy(ob_a.at[pl.ds(0, _DROWS)],
                            num_sp.at[pl.ds(t * _ROWS_T + k * _DROWS,
                                            _DROWS)])
            return carry

        lax.fori_loop(0, _ROWS_T // _DROWS, zcopy, 0)
        plsc.subcore_barrier()
        pltpu.sync_copy(att2.at[2 * r + c], attv)
        atts = [attv[16 * j:16 * (j + 1)] for j in range(8)]
        atts2 = [0.2 * a for a in atts]

        def issue(k, ib, rs, rd, sem):
            pltpu.sync_copy(edges7.at[r, c, t * _NCHUNK + k], ib)
            pltpu.async_copy(xs_flat.at[ib.at[0]], rs, sem)
            pltpu.async_copy(xd_flat.at[ib.at[1]], rd, sem)

        def wait(ib, rs, rd, sem):
            pltpu.make_async_copy(xs_flat.at[ib.at[0]], rs, sem).wait()
            pltpu.make_async_copy(xd_flat.at[ib.at[1]], rd, sem).wait()

        def compute(rs, rd, ob):
            @plsc.parallel_loop(0, _EC, unroll=4)
            def _edge(e):
                svals = [rs[e, 16 * j:16 * (j + 1)] for j in range(8)]
                parts = []
                for p in range(2):
                    acc = None
                    for j in range(4 * p, 4 * p + 4):
                        x = svals[j] + rd[e, 16 * j:16 * (j + 1)]
                        av = jnp.where(x >= 0.0, atts[j], atts2[j])
                        acc = x * av if acc is None else acc + x * av
                    parts.append(acc)
                exv = jnp.exp(jnp.full((16,), jnp.sum(parts[0] + parts[1])))
                for j in range(8):
                    ob[e, 16 * j:16 * (j + 1)] = exv * svals[j]
                ob[e, 128:144] = jnp.where(mask0, exv, z16)

        issue(0, ib0, rs_a, rd_a, sem_a)

        def quad(i, carry):
            for j in range(4):
                k = 4 * i + j
                nb = (j + 1) % 4
                issue(jnp.minimum(k + 1, _NCHUNK - 1),
                      ibs[nb], rss[nb % 2], rds[nb % 2], sems[nb % 2])
                wait(ibs[j], rss[j % 2], rds[j % 2], sems[j % 2])
                sdesc = pltpu.make_async_copy(obs[j % 2],
                                              num_sp.at[ibs[j].at[2]],
                                              sss[j % 2])
                if j >= 2:
                    sdesc.wait()
                else:
                    @pl.when(i > 0)
                    def _():
                        sdesc.wait()

                compute(rss[j % 2], rds[j % 2], obs[j % 2])
                pltpu.async_copy(obs[j % 2], num_sp.at[ibs[j].at[2]],
                                 sss[j % 2], add=True)
            return carry

        lax.fori_loop(0, _NCHUNK // 4, quad, 0)
        wait(ib0, rs_a, rd_a, sem_a)
        pltpu.make_async_copy(ob_a, num_sp.at[ib2.at[2]], ss_a).wait()
        pltpu.make_async_copy(ob_b, num_sp.at[ib3.at[2]], ss_b).wait()
        plsc.subcore_barrier()

        def dump(k, carry):
            rb = t * _ROWS_T + k * _DROWS
            pltpu.sync_copy(num_sp.at[pl.ds(rb, _DROWS)],
                            ob_a.at[pl.ds(0, _DROWS)])

            def nrow(i, ncarry):
                den = ob_a[i, 128:144][0]
                inv = 1.0 / (jnp.full((16,), den) + 1e-16)
                for j in range(8):
                    nbuf[i, 16 * j:16 * (j + 1)] = \
                        ob_a[i, 16 * j:16 * (j + 1)] * inv
                return ncarry

            lax.fori_loop(0, _DROWS, nrow, 0)
            pltpu.sync_copy(nbuf,
                            out_flat.at[pl.ds((2 * r + c) * N + rb, _DROWS)])
            return carry

        lax.fori_loop(0, _ROWS_T // _DROWS, dump, 0)
        plsc.subcore_barrier()
        return carry0

    lax.fori_loop(0, 5, relation, 0)


def _edge_call(xs_flat, xd_flat, edges7, att2):
    mesh = plsc.VectorSubcoreMesh(core_axis_name="c", subcore_axis_name="s")
    f = pl.kernel(
        _edge_body,
        out_type=jax.ShapeDtypeStruct((10 * N, D), F32),
        mesh=mesh,
        compiler_params=pltpu.CompilerParams(use_tc_tiling_on_sc=False, needs_layout_passes=False),
        scratch_types=[
            pltpu.VMEM_SHARED((_NSP, _W), F32),
            pltpu.VMEM((3, _EC), jnp.int32),
            pltpu.VMEM((3, _EC), jnp.int32),
            pltpu.VMEM((3, _EC), jnp.int32),
            pltpu.VMEM((3, _EC), jnp.int32),
            pltpu.VMEM((_EC, D), F32),
            pltpu.VMEM((_EC, D), F32),
            pltpu.VMEM((_EC, D), F32),
            pltpu.VMEM((_EC, D), F32),
            pltpu.VMEM((_EC, _W), F32),
            pltpu.VMEM((_EC, _W), F32),
            pltpu.VMEM((D,), F32),
            pltpu.VMEM((_DROWS, D), F32),
            pltpu.SemaphoreType.DMA,
            pltpu.SemaphoreType.DMA,
            pltpu.SemaphoreType.DMA,
            pltpu.SemaphoreType.DMA,
        ],
    )
    return f(xs_flat, xd_flat, edges7, att2)


def _combine_body(num, pep, mhc, tcr, rb, wp, bp, h_all):
    def rel(r):
        return 0.5 * (num[r, 0] + num[r, 1]) + rb[r][None, :]

    def elu(x):
        return jnp.where(x > 0.0, x, jnp.exp(jnp.minimum(x, 0.0)) - 1.0)

    out_mhc = rel(0)
    out_tcr = 0.5 * (rel(1) + rel(2))
    out_pep = 0.5 * (rel(3) + rel(4))
    h_pep = elu(out_pep + pep[...])
    h_all[0] = jnp.dot(h_pep, wp[...], preferred_element_type=F32) \
        + bp[0][None, :]
    h_all[1] = elu(out_mhc + mhc[...])
    h_all[2] = elu(out_tcr + tcr[...])


def _combine_call(num4, pep, mhc, tcr, rel_bias, wp, bp):
    grid = (N // _NBLK,)
    node_spec = pl.BlockSpec((_NBLK, D), lambda i: (i, 0))
    return pl.pallas_call(
        _combine_body,
        grid=grid,
        in_specs=[
            pl.BlockSpec((5, 2, _NBLK, D), lambda i: (0, 0, i, 0)),
            node_spec, node_spec, node_spec,
            pl.BlockSpec((5, D), lambda i: (0, 0)),
            pl.BlockSpec((D, D), lambda i: (0, 0)),
            pl.BlockSpec((1, D), lambda i: (0, 0)),
        ],
        out_specs=pl.BlockSpec((3, _NBLK, D), lambda i: (0, i, 0)),
        out_shape=jax.ShapeDtypeStruct((3, N, D), F32),
    )(num4, pep, mhc, tcr, rel_bias, wp, bp)


_GC = 128
_GPW = 3 * B // 32 // _GC


def _tgather_body(table, tidx, out, ibuf, rbuf):
    c = lax.axis_index("c")
    t = lax.axis_index("s")
    wid = t * 2 + c

    def chunk(k, carry):
        base = wid * (_GPW * _GC) + k * _GC
        pltpu.sync_copy(tidx.at[pl.ds(base, _GC)], ibuf)
        pltpu.sync_copy(table.at[ibuf], rbuf)
        pltpu.sync_copy(rbuf, out.at[pl.ds(base, _GC)])
        return carry

    lax.fori_loop(0, _GPW, chunk, 0)


def _tgather_call(table_flat, tidx_flat):
    mesh = plsc.VectorSubcoreMesh(core_axis_name="c", subcore_axis_name="s")
    f = pl.kernel(
        _tgather_body,
        out_type=jax.ShapeDtypeStruct((3 * B, D), F32),
        mesh=mesh,
        compiler_params=pltpu.CompilerParams(use_tc_tiling_on_sc=False, needs_layout_passes=False),
        scratch_types=[
            pltpu.VMEM((_GC,), jnp.int32),
            pltpu.VMEM((_GC, D), F32),
        ],
    )
    return f(table_flat, tidx_flat)


_BBLK = 1024


def _mlp_body(hb, w1pm, b1pm, w2pm, b2pm, wpm, w1mt, b1mt, w2mt, b2mt,
              w1df, b1df, wdf2, scb, out):
    hpb, hmb, htb = hb[0], hb[1], hb[2]

    def mm(x, w):
        return jnp.dot(x, w, preferred_element_type=F32)

    v = jnp.maximum(mm(hpb, w1pm[:D]) + mm(hmb, w1pm[D:]) + b1pm[0][None, :],
                    0.0)
    v_pm = mm(v, w2pm[...]) + b2pm[0][None, :]
    logit_pm = jnp.sum(v_pm * wpm[0][None, :], axis=1) + scb[0, 0]
    u = jnp.maximum(mm(hmb, w1mt[:D]) + mm(htb, w1mt[D:]) + b1mt[0][None, :],
                    0.0)
    v_mt = mm(u, w2mt[...]) + b2mt[0][None, :]
    z = v_pm * v_mt
    z1 = jnp.maximum(mm(z, w1df[...]) + b1df[0][None, :], 0.0)
    logit_pmt = jnp.sum(z1 * wdf2[0][None, :], axis=1) + scb[1, 0]
    out[0] = logit_pm
    out[1] = logit_pmt


def _mlp_call(hb3, p):
    grid = (B // _BBLK,)

    def full(shape):
        nd = len(shape)
        return pl.BlockSpec(shape, lambda i, _n=nd: (0,) * _n)

    w1pm = p['f_pm']['l1']['W'].T
    b1pm = p['f_pm']['l1']['b'][None, :]
    w2pm = p['f_pm']['l2']['W'].T
    b2pm = p['f_pm']['l2']['b'][None, :]
    wpm = p['w_pm']['W']
    w1mt = p['f_mt']['l1']['W'].T
    b1mt = p['f_mt']['l1']['b'][None, :]
    w2mt = p['f_mt']['l2']['W'].T
    b2mt = p['f_mt']['l2']['b'][None, :]
    w1df = p['f_dmf']['l1']['W'].T
    b1df = p['f_dmf']['l1']['b'][None, :]
    wdf2 = p['f_dmf']['l2']['W']
    scb = jnp.stack([
        jnp.pad(p['w_pm']['b'], (0, D - 1)),
        jnp.pad(p['f_dmf']['l2']['b'], (0, D - 1)),
    ])
    return pl.pallas_call(
        _mlp_body,
        grid=grid,
        in_specs=[
            pl.BlockSpec((3, _BBLK, D), lambda i: (0, i, 0)),
            full((2 * D, D)), full((1, D)), full((D, D)), full((1, D)),
            full((1, D)),
            full((2 * D, D)), full((1, D)), full((D, D)), full((1, D)),
            full((D, D)), full((1, D)), full((1, D)), full((2, D)),
        ],
        out_specs=pl.BlockSpec((2, _BBLK), lambda i: (0, i)),
        out_shape=jax.ShapeDtypeStruct((2, B), F32),
    )(hb3, w1pm, b1pm, w2pm, b2pm, wpm, w1mt, b1mt, w2mt, b2mt,
      w1df, b1df, wdf2, scb)


def kernel(params, edge_binds, edge_presents_to, edge_contacts,
           edge_bound_by, edge_contacted_by, triplet_idx):
    p = params
    rels = p['rels']

    def heads_t(w):
        return w.reshape(H, HID, D).transpose(0, 2, 1)

    wl = jnp.stack([heads_t(rels[r]['lin_l']['W']) for r in _RELS])
    wr = jnp.stack([heads_t(rels[r]['lin_r']['W']) for r in _RELS])
    bl = jnp.stack([rels[r]['lin_l']['b'] for r in _RELS]).reshape(10, D)
    br = jnp.stack([rels[r]['lin_r']['b'] for r in _RELS]).reshape(10, D)
    att2 = jnp.stack([rels[r]['att'] for r in _RELS]).reshape(10, D)
    rel_bias = jnp.stack([rels[r]['bias'] for r in _RELS])

    xs4, xd4 = _proj_call(p['emb_pep'], p['emb_mhc'], p['emb_tcr'],
                          wl, bl, wr, br)
    xs_flat = xs4.reshape(10 * N, D)
    xd_flat = xd4.reshape(10 * N, D)

    edges = [edge_binds, edge_presents_to, edge_contacts, edge_bound_by,
             edge_contacted_by]
    e_raw = jnp.stack(edges)
    offs = (jnp.arange(5, dtype=jnp.int32) * 2)[:, None, None]
    head = jnp.arange(2, dtype=jnp.int32)[None, :, None]
    src_flat = (offs + head) * N + e_raw[:, None, 0, :]
    dst_flat = (offs + head) * N + e_raw[:, None, 1, :]
    dst_loc = jnp.broadcast_to(e_raw[:, None, 1, :], (5, 2, E))
    base = ((offs + head) * N).astype(jnp.int32)
    park = jnp.arange(_EPAD, dtype=jnp.int32)[None, None, :]
    gpad = jnp.broadcast_to(base + park % 128, (5, 2, _EPAD))
    spad = jnp.broadcast_to(N + park % 16, (5, 2, _EPAD))
    src_flat = jnp.concatenate([src_flat, gpad], axis=-1)
    dst_flat = jnp.concatenate([dst_flat, gpad], axis=-1)
    dst_loc = jnp.concatenate([dst_loc, spad], axis=-1)
    edges7 = jnp.stack([x.reshape(5, 2, 16 * _NCHUNK, _EC)
                        for x in (src_flat, dst_flat, dst_loc)], axis=3)

    num_flat = _edge_call(xs_flat, xd_flat, edges7, att2)
    num4 = num_flat.reshape(5, 2, N, D)

    h_all = _combine_call(num4, p['emb_pep'], p['emb_mhc'], p['emb_tcr'],
                          rel_bias, p['proj_pep']['W'].T,
                          p['proj_pep']['b'][None, :])
    table_flat = h_all.reshape(3 * N, D)

    tidx_flat = (triplet_idx
                 + (jnp.arange(3, dtype=jnp.int32) * N)[:, None]).reshape(-1)
    hb_flat = _tgather_call(table_flat, tidx_flat)
    hb3 = hb_flat.reshape(3, B, D)

    return _mlp_call(hb3, p)


# speedup vs baseline: 2.1021x; 1.1741x over previous
import functools

import jax
import jax.numpy as jnp
from jax import lax
from jax.experimental import pallas as pl
from jax.experimental.pallas import tpu as pltpu
from jax.experimental.pallas import tpu_sc as plsc

N = 10000
E = 160000
D = 128
HID = 128
H = 2
B = 16384
F32 = jnp.float32

_RELS = ['binds', 'presents_to', 'contacts', 'bound_by', 'contacted_by']
_SRC = [0, 1, 0, 1, 2]
_DST = [1, 2, 2, 0, 0]


_NBLK = 1000


def _proj_body(pep, mhc, tcr, wl, bl, wr, br, xs, xd):
    embs = (pep[...], mhc[...], tcr[...])
    for r in range(5):
        for h in range(2):
            xs[r, h] = (jnp.dot(embs[_SRC[r]], wl[r, h],
                                preferred_element_type=F32)
                        + bl[2 * r + h][None, :])
            xd[r, h] = (jnp.dot(embs[_DST[r]], wr[r, h],
                                preferred_element_type=F32)
                        + br[2 * r + h][None, :])


def _proj_call(pep, mhc, tcr, wl, bl, wr, br):
    grid = (N // _NBLK,)
    node_spec = pl.BlockSpec((_NBLK, D), lambda i: (i, 0))
    full4 = pl.BlockSpec((5, 2, D, D), lambda i: (0, 0, 0, 0))
    full2 = pl.BlockSpec((10, D), lambda i: (0, 0))
    out_spec = pl.BlockSpec((5, 2, _NBLK, D), lambda i: (0, 0, i, 0))
    return pl.pallas_call(
        _proj_body,
        grid=grid,
        in_specs=[node_spec, node_spec, node_spec, full4, full2, full4, full2],
        out_specs=[out_spec, out_spec],
        out_shape=[jax.ShapeDtypeStruct((5, 2, N, D), F32)] * 2,
    )(pep, mhc, tcr, wl, bl, wr, br)


_EC = 40
_NCHUNK = 252
_EPAD = 16 * _NCHUNK * _EC - E
_ROWS_T = N // 16
_DROWS = 25
_W = 144
_NSP = N + 16
_PB = 48


def _edge_body(xs_flat, xd_flat, edges7, att2, out_flat,
               num_sp, ib0, ib1, ib2, ib3, rs_a, rs_b, rd_a, rd_b,
               ob_a, ob_b, attv, nbuf,
               sem_a, sem_b, ss_a, ss_b, sip_a, sip_b):
    c = lax.axis_index("c")
    t = lax.axis_index("s")
    z16 = jnp.zeros((16,), F32)
    mask0 = lax.iota(jnp.int32, 16) == 0
    iota16 = lax.iota(jnp.int32, 16)
    ibs = [ib0, ib1, ib2, ib3]
    rss = [rs_a, rs_b]
    rds = [rd_a, rd_b]
    obs = [ob_a, ob_b]
    sems = [sem_a, sem_b]
    sss = [ss_a, ss_b]
    sips = [sip_a, sip_b]

    def relation(r, carry0):
        def zrow(i, carry):
            for j in range(_W // 16):
                ob_a[i, 16 * j:16 * (j + 1)] = z16
            return carry

        lax.fori_loop(0, _DROWS, zrow, 0)

        def zcopy(k, carry):
            pltpu.sync_copy(ob_a.at[pl.ds(0, _DROWS)],
                            num_sp.at[pl.ds(t * _ROWS_T + k * _DROWS,
                                            _DROWS)])
            return carry

        lax.fori_loop(0, _ROWS_T // _DROWS, zcopy, 0)
        plsc.subcore_barrier()
        pltpu.sync_copy(att2.at[2 * r + c], attv)
        atts = [attv[16 * j:16 * (j + 1)] for j in range(8)]
        atts2 = [0.2 * a for a in atts]

        def fetch_idx(k, ib, sem):
            pltpu.async_copy(edges7.at[r, c, t * _NCHUNK + k], ib, sem)

        def wait_idx(ib, sem):
            pltpu.make_async_copy(edges7.at[r, c, 0], ib, sem).wait()

        def gathers(ib, rs, rd, sem):
            pltpu.async_copy(xs_flat.at[ib.at[0]], rs, sem)
            pltpu.async_copy(xd_flat.at[ib.at[1]], rd, sem)

        def wait_g(ib, rs, rd, sem):
            pltpu.make_async_copy(xs_flat.at[ib.at[0]], rs, sem).wait()
            pltpu.make_async_copy(xd_flat.at[ib.at[1]], rd, sem).wait()

        def compute(rs, rd, ob):
            @plsc.parallel_loop(0, _EC, unroll=4)
            def _edge(e):
                svals = [rs[e, 16 * j:16 * (j + 1)] for j in range(8)]
                parts = []
                for p in range(2):
                    acc = None
                    for j in range(4 * p, 4 * p + 4):
                        x = svals[j] + rd[e, 16 * j:16 * (j + 1)]
                        av = jnp.where(x >= 0.0, atts[j], atts2[j])
                        acc = x * av if acc is None else acc + x * av
                    parts.append(acc)
                exv = jnp.exp(jnp.full((16,), jnp.sum(parts[0] + parts[1])))
                for j in range(8):
                    ob[e, 16 * j:16 * (j + 1)] = exv * svals[j]
                ob[e, 128:144] = jnp.where(mask0, exv, z16)

        fetch_idx(0, ib0, sip_a)
        fetch_idx(1, ib1, sip_b)
        wait_idx(ib0, sip_a)
        gathers(ib0, rs_a, rd_a, sem_a)

        def quad(i, carry):
            for j in range(4):
                k = 4 * i + j
                nb = (j + 1) % 4
                wait_idx(ibs[nb], sips[(j + 1) % 2])
                gathers(ibs[nb], rss[nb % 2], rds[nb % 2], sems[nb % 2])
                sdesc = pltpu.make_async_copy(obs[j % 2],
                                              num_sp.at[ibs[j].at[2]],
                                              sss[j % 2])
                if j >= 2:
                    sdesc.wait()
                else:
                    @pl.when(i > 0)
                    def _():
                        sdesc.wait()

                fetch_idx(jnp.minimum(k + 2, _NCHUNK - 1),
                          ibs[(j + 2) % 4], sips[j % 2])
                wait_g(ibs[j], rss[j % 2], rds[j % 2], sems[j % 2])
                compute(rss[j % 2], rds[j % 2], obs[j % 2])
                pltpu.async_copy(obs[j % 2], num_sp.at[ibs[j].at[2]],
                                 sss[j % 2], add=True)
            return carry

        lax.fori_loop(0, _NCHUNK // 4, quad, 0)
        wait_idx(ib1, sip_b)
        wait_g(ib0, rs_a, rd_a, sem_a)
        pltpu.make_async_copy(ob_a, num_sp.at[ib2.at[2]], ss_a).wait()
        pltpu.make_async_copy(ob_b, num_sp.at[ib3.at[2]], ss_b).wait()
        plsc.subcore_barrier()

        def dump(k, carry):
            rb = t * _ROWS_T + k * _DROWS
            pltpu.sync_copy(num_sp.at[pl.ds(rb, _DROWS)],
                            ob_a.at[pl.ds(0, _DROWS)])

            def nrow(i, ncarry):
                den = ob_a[i, 128:144][0]
                inv = 1.0 / (jnp.full((16,), den) + 1e-16)
                for j in range(8):
                    nbuf[i, 16 * j:16 * (j + 1)] = \
                        ob_a[i, 16 * j:16 * (j + 1)] * inv
                return ncarry

            lax.fori_loop(0, _DROWS, nrow, 0)
            pltpu.sync_copy(nbuf,
                            out_flat.at[pl.ds((2 * r + c) * N + rb, _DROWS)])
            return carry

        lax.fori_loop(0, _ROWS_T // _DROWS, dump, 0)
        plsc.subcore_barrier()
        return carry0

    lax.fori_loop(0, 5, relation, 0)


def _edge_call(xs_flat, xd_flat, edges7, att2):
    mesh = plsc.VectorSubcoreMesh(core_axis_name="c", subcore_axis_name="s")
    f = pl.kernel(
        _edge_body,
        out_type=jax.ShapeDtypeStruct((10 * N, D), F32),
        mesh=mesh,
        compiler_params=pltpu.CompilerParams(use_tc_tiling_on_sc=False, needs_layout_passes=False),
        scratch_types=[
            pltpu.VMEM_SHARED((_NSP, _W), F32),
            pltpu.VMEM((3, _EC), jnp.int32),
            pltpu.VMEM((3, _EC), jnp.int32),
            pltpu.VMEM((3, _EC), jnp.int32),
            pltpu.VMEM((3, _EC), jnp.int32),
            pltpu.VMEM((_EC, D), F32),
            pltpu.VMEM((_EC, D), F32),
            pltpu.VMEM((_EC, D), F32),
            pltpu.VMEM((_EC, D), F32),
            pltpu.VMEM((_EC, _W), F32),
            pltpu.VMEM((_EC, _W), F32),
            pltpu.VMEM((D,), F32),
            pltpu.VMEM((_DROWS, D), F32),
            pltpu.SemaphoreType.DMA,
            pltpu.SemaphoreType.DMA,
            pltpu.SemaphoreType.DMA,
            pltpu.SemaphoreType.DMA,
            pltpu.SemaphoreType.DMA,
            pltpu.SemaphoreType.DMA,
        ],
    )
    return f(xs_flat, xd_flat, edges7, att2)


def _combine_body(num, pep, mhc, tcr, rb, wp, bp, h_all):
    def rel(r):
        return 0.5 * (num[r, 0] + num[r, 1]) + rb[r][None, :]

    def elu(x):
        return jnp.where(x > 0.0, x, jnp.exp(jnp.minimum(x, 0.0)) - 1.0)

    out_mhc = rel(0)
    out_tcr = 0.5 * (rel(1) + rel(2))
    out_pep = 0.5 * (rel(3) + rel(4))
    h_pep = elu(out_pep + pep[...])
    h_all[0] = jnp.dot(h_pep, wp[...], preferred_element_type=F32) \
        + bp[0][None, :]
    h_all[1] = elu(out_mhc + mhc[...])
    h_all[2] = elu(out_tcr + tcr[...])


def _combine_call(num4, pep, mhc, tcr, rel_bias, wp, bp):
    grid = (N // _NBLK,)
    node_spec = pl.BlockSpec((_NBLK, D), lambda i: (i, 0))
    return pl.pallas_call(
        _combine_body,
        grid=grid,
        in_specs=[
            pl.BlockSpec((5, 2, _NBLK, D), lambda i: (0, 0, i, 0)),
            node_spec, node_spec, node_spec,
            pl.BlockSpec((5, D), lambda i: (0, 0)),
            pl.BlockSpec((D, D), lambda i: (0, 0)),
            pl.BlockSpec((1, D), lambda i: (0, 0)),
        ],
        out_specs=pl.BlockSpec((3, _NBLK, D), lambda i: (0, i, 0)),
        out_shape=jax.ShapeDtypeStruct((3, N, D), F32),
    )(num4, pep, mhc, tcr, rel_bias, wp, bp)


_GC = 128
_GPW = 3 * B // 32 // _GC


def _tgather_body(table, tidx, out, ibuf, rbuf):
    c = lax.axis_index("c")
    t = lax.axis_index("s")
    wid = t * 2 + c

    def chunk(k, carry):
        base = wid * (_GPW * _GC) + k * _GC
        pltpu.sync_copy(tidx.at[pl.ds(base, _GC)], ibuf)
        pltpu.sync_copy(table.at[ibuf], rbuf)
        pltpu.sync_copy(rbuf, out.at[pl.ds(base, _GC)])
        return carry

    lax.fori_loop(0, _GPW, chunk, 0)


def _tgather_call(table_flat, tidx_flat):
    mesh = plsc.VectorSubcoreMesh(core_axis_name="c", subcore_axis_name="s")
    f = pl.kernel(
        _tgather_body,
        out_type=jax.ShapeDtypeStruct((3 * B, D), F32),
        mesh=mesh,
        compiler_params=pltpu.CompilerParams(use_tc_tiling_on_sc=False, needs_layout_passes=False),
        scratch_types=[
            pltpu.VMEM((_GC,), jnp.int32),
            pltpu.VMEM((_GC, D), F32),
        ],
    )
    return f(table_flat, tidx_flat)


_BBLK = 1024


def _mlp_body(hb, w1pm, b1pm, w2pm, b2pm, wpm, w1mt, b1mt, w2mt, b2mt,
              w1df, b1df, wdf2, scb, out):
    hpb, hmb, htb = hb[0], hb[1], hb[2]

    def mm(x, w):
        return jnp.dot(x, w, preferred_element_type=F32)

    v = jnp.maximum(mm(hpb, w1pm[:D]) + mm(hmb, w1pm[D:]) + b1pm[0][None, :],
                    0.0)
    v_pm = mm(v, w2pm[...]) + b2pm[0][None, :]
    logit_pm = jnp.sum(v_pm * wpm[0][None, :], axis=1) + scb[0, 0]
    u = jnp.maximum(mm(hmb, w1mt[:D]) + mm(htb, w1mt[D:]) + b1mt[0][None, :],
                    0.0)
    v_mt = mm(u, w2mt[...]) + b2mt[0][None, :]
    z = v_pm * v_mt
    z1 = jnp.maximum(mm(z, w1df[...]) + b1df[0][None, :], 0.0)
    logit_pmt = jnp.sum(z1 * wdf2[0][None, :], axis=1) + scb[1, 0]
    out[0] = logit_pm
    out[1] = logit_pmt


def _mlp_call(hb3, p):
    grid = (B // _BBLK,)

    def full(shape):
        nd = len(shape)
        return pl.BlockSpec(shape, lambda i, _n=nd: (0,) * _n)

    w1pm = p['f_pm']['l1']['W'].T
    b1pm = p['f_pm']['l1']['b'][None, :]
    w2pm = p['f_pm']['l2']['W'].T
    b2pm = p['f_pm']['l2']['b'][None, :]
    wpm = p['w_pm']['W']
    w1mt = p['f_mt']['l1']['W'].T
    b1mt = p['f_mt']['l1']['b'][None, :]
    w2mt = p['f_mt']['l2']['W'].T
    b2mt = p['f_mt']['l2']['b'][None, :]
    w1df = p['f_dmf']['l1']['W'].T
    b1df = p['f_dmf']['l1']['b'][None, :]
    wdf2 = p['f_dmf']['l2']['W']
    scb = jnp.stack([
        jnp.pad(p['w_pm']['b'], (0, D - 1)),
        jnp.pad(p['f_dmf']['l2']['b'], (0, D - 1)),
    ])
    return pl.pallas_call(
        _mlp_body,
        grid=grid,
        in_specs=[
            pl.BlockSpec((3, _BBLK, D), lambda i: (0, i, 0)),
            full((2 * D, D)), full((1, D)), full((D, D)), full((1, D)),
            full((1, D)),
            full((2 * D, D)), full((1, D)), full((D, D)), full((1, D)),
            full((D, D)), full((1, D)), full((1, D)), full((2, D)),
        ],
        out_specs=pl.BlockSpec((2, _BBLK), lambda i: (0, i)),
        out_shape=jax.ShapeDtypeStruct((2, B), F32),
    )(hb3, w1pm, b1pm, w2pm, b2pm, wpm, w1mt, b1mt, w2mt, b2mt,
      w1df, b1df, wdf2, scb)


def kernel(params, edge_binds, edge_presents_to, edge_contacts,
           edge_bound_by, edge_contacted_by, triplet_idx):
    p = params
    rels = p['rels']

    def heads_t(w):
        return w.reshape(H, HID, D).transpose(0, 2, 1)

    wl = jnp.stack([heads_t(rels[r]['lin_l']['W']) for r in _RELS])
    wr = jnp.stack([heads_t(rels[r]['lin_r']['W']) for r in _RELS])
    bl = jnp.stack([rels[r]['lin_l']['b'] for r in _RELS]).reshape(10, D)
    br = jnp.stack([rels[r]['lin_r']['b'] for r in _RELS]).reshape(10, D)
    att2 = jnp.stack([rels[r]['att'] for r in _RELS]).reshape(10, D)
    rel_bias = jnp.stack([rels[r]['bias'] for r in _RELS])

    xs4, xd4 = _proj_call(p['emb_pep'], p['emb_mhc'], p['emb_tcr'],
                          wl, bl, wr, br)
    xs_flat = xs4.reshape(10 * N, D)
    xd_flat = xd4.reshape(10 * N, D)

    edges = [edge_binds, edge_presents_to, edge_contacts, edge_bound_by,
             edge_contacted_by]
    e_raw = jnp.stack(edges)
    offs = (jnp.arange(5, dtype=jnp.int32) * 2)[:, None, None]
    head = jnp.arange(2, dtype=jnp.int32)[None, :, None]
    src_flat = (offs + head) * N + e_raw[:, None, 0, :]
    dst_flat = (offs + head) * N + e_raw[:, None, 1, :]
    dst_loc = jnp.broadcast_to(e_raw[:, None, 1, :], (5, 2, E))
    base = ((offs + head) * N).astype(jnp.int32)
    park = jnp.arange(_EPAD, dtype=jnp.int32)[None, None, :]
    gpad = jnp.broadcast_to(base + park % 128, (5, 2, _EPAD))
    spad = jnp.broadcast_to(N + park % 16, (5, 2, _EPAD))
    src_flat = jnp.concatenate([src_flat, gpad], axis=-1)
    dst_flat = jnp.concatenate([dst_flat, gpad], axis=-1)
    dst_loc = jnp.concatenate([dst_loc, spad], axis=-1)
    edges7 = jnp.stack([x.reshape(5, 2, 16 * _NCHUNK, _EC)
                        for x in (src_flat, dst_flat, dst_loc)], axis=3)

    num_flat = _edge_call(xs_flat, xd_flat, edges7, att2)
    num4 = num_flat.reshape(5, 2, N, D)

    h_all = _combine_call(num4, p['emb_pep'], p['emb_mhc'], p['emb_tcr'],
                          rel_bias, p['proj_pep']['W'].T,
                          p['proj_pep']['b'][None, :])
    table_flat = h_all.reshape(3 * N, D)

    tidx_flat = (triplet_idx
                 + (jnp.arange(3, dtype=jnp.int32) * N)[:, None]).reshape(-1)
    hb_flat = _tgather_call(table_flat, tidx_flat)
    hb3 = hb_flat.reshape(3, B, D)

    return _mlp_call(hb3, p)
